# Initial kernel scaffold; baseline (speedup 1.0000x reference)
import jax
import jax.numpy as jnp
from jax.experimental import pallas as pl


def kernel(x, weight):
    raise NotImplementedError("write your pallas kernel here")


# speedup vs baseline: 1.5741x; 1.5741x over previous
import functools

import jax
import jax.numpy as jnp
from jax import lax
from jax.experimental import pallas as pl
from jax.experimental.pallas import tpu as pltpu
from jax.experimental.pallas import tpu_sc as plsc

_NC = 2
_NS = 16
_NW = _NC * _NS
_WIN = 128


def _sc_gather(idx_flat, weight):
    n = idx_flat.shape[0]
    dim = weight.shape[1]
    b_per_w = n // _NW
    n_win = b_per_w // _WIN
    mesh = plsc.VectorSubcoreMesh(core_axis_name="c", subcore_axis_name="s")

    @functools.partial(
        pl.kernel,
        mesh=mesh,
        out_type=jax.ShapeDtypeStruct((n, dim), jnp.float32),
        scratch_types=[
            pltpu.VMEM((_WIN,), jnp.int32),
            pltpu.VMEM((_WIN, dim), jnp.float32),
            pltpu.SemaphoreType.DMA,
        ],
        compiler_params=pltpu.CompilerParams(use_tc_tiling_on_sc=False),
    )
    def gather_kernel(idx_hbm, table_hbm, out_hbm, idx_v, rows_v, sem):
        wid = lax.axis_index("s") * _NC + lax.axis_index("c")
        base = wid * b_per_w

        @pl.loop(0, n_win)
        def _(g):
            off = base + g * _WIN
            pltpu.sync_copy(idx_hbm.at[pl.ds(off, _WIN)], idx_v)
            pltpu.async_copy(table_hbm.at[idx_v], rows_v, sem).wait()
            pltpu.sync_copy(rows_v, out_hbm.at[pl.ds(off, _WIN)])

    return gather_kernel(idx_flat, weight)


def kernel(x, weight):
    b, s = x.shape
    idx_flat = x.reshape(b * s).astype(jnp.int32)
    out = _sc_gather(idx_flat, weight)
    return out.reshape(b, s, weight.shape[1])


# speedup vs baseline: 1.8768x; 1.1923x over previous
import functools

import jax
import jax.numpy as jnp
from jax import lax
from jax.experimental import pallas as pl
from jax.experimental.pallas import tpu as pltpu
from jax.experimental.pallas import tpu_sc as plsc

_NC = 2
_NS = 16
_NW = _NC * _NS
_G = 128
_GPW = 4
_WIN = _G * _GPW


def _sc_gather(idx2d, weight):
    n_rows = idx2d.shape[0]
    n = n_rows * _G
    dim = weight.shape[1]
    rows_per_w = n_rows // _NW
    n_win = rows_per_w // _GPW
    mesh = plsc.VectorSubcoreMesh(core_axis_name="c", subcore_axis_name="s")

    @functools.partial(
        pl.kernel,
        mesh=mesh,
        out_type=jax.ShapeDtypeStruct((n, dim), jnp.float32),
        scratch_types=[
            pltpu.VMEM((2, _GPW, _G), jnp.int32),
            pltpu.VMEM((2, _WIN, dim), jnp.float32),
            pltpu.SemaphoreType.DMA((2,)),
            pltpu.SemaphoreType.DMA((2,)),
            pltpu.SemaphoreType.DMA((2,)),
        ],
        compiler_params=pltpu.CompilerParams(use_tc_tiling_on_sc=False),
    )
    def gather_kernel(idx_hbm, table_hbm, out_hbm, idx_v, rows_v, sem_i,
                      sem_g, sem_o):
        wid = lax.axis_index("s") * _NC + lax.axis_index("c")
        idx_base = wid * rows_per_w
        out_base = wid * rows_per_w * _G

        def issue_idx(w, b):
            pltpu.async_copy(
                idx_hbm.at[pl.ds(idx_base + w * _GPW, _GPW)],
                idx_v.at[b], sem_i.at[b])

        def wait_idx(b):
            pltpu.make_async_copy(
                idx_hbm.at[pl.ds(0, _GPW)], idx_v.at[b], sem_i.at[b]).wait()

        def issue_gathers(b):
            for j in range(_GPW):
                pltpu.async_copy(
                    table_hbm.at[idx_v.at[b, j]],
                    rows_v.at[b, pl.ds(j * _G, _G)], sem_g.at[b])

        def wait_gathers(b):
            pltpu.make_async_copy(
                table_hbm.at[pl.ds(0, _WIN)], rows_v.at[b], sem_g.at[b]).wait()

        def issue_out(w, b):
            pltpu.async_copy(
                rows_v.at[b], out_hbm.at[pl.ds(out_base + w * _WIN, _WIN)],
                sem_o.at[b])

        def wait_out(b):
            pltpu.make_async_copy(
                rows_v.at[b], out_hbm.at[pl.ds(0, _WIN)], sem_o.at[b]).wait()

        issue_idx(0, 0)
        issue_idx(1, 1)
        wait_idx(0)
        issue_gathers(0)

        @pl.loop(0, n_win - 2, step=2)
        def _(g):
            wait_idx(1)

            @pl.when(g > 0)
            def _():
                wait_out(1)

            issue_gathers(1)
            wait_gathers(0)
            issue_out(g, 0)
            issue_idx(g + 2, 0)

            wait_idx(0)
            wait_out(0)
            issue_gathers(0)
            wait_gathers(1)
            issue_out(g + 1, 1)
            issue_idx(g + 3, 1)

        wait_idx(1)
        wait_out(1)
        issue_gathers(1)
        wait_gathers(0)
        issue_out(n_win - 2, 0)
        wait_gathers(1)
        issue_out(n_win - 1, 1)
        wait_out(0)
        wait_out(1)

    return gather_kernel(idx2d, weight)


def kernel(x, weight):
    b, s = x.shape
    idx2d = x.reshape(b * s // _G, _G).astype(jnp.int32)
    out = _sc_gather(idx2d, weight)
    return out.reshape(b, s, weight.shape[1])


# speedup vs baseline: 1.8785x; 1.0009x over previous
import functools

import jax
import jax.numpy as jnp
from jax import lax
from jax.experimental import pallas as pl
from jax.experimental.pallas import tpu as pltpu
from jax.experimental.pallas import tpu_sc as plsc

_NC = 2
_NS = 16
_NW = _NC * _NS
_G = 128
_BPW = 16


def _sc_gather(idx_flat, weight, batch, seq):
    dim = weight.shape[1]
    n = idx_flat.shape[0]
    ids_per_worker = n // _NW
    win_ids = _BPW * seq
    n_win = ids_per_worker // win_ids
    batches_per_worker = batch // _NW
    chunks = []
    off = 0
    while off < win_ids:
        c = min(_G, win_ids - off)
        chunks.append((off, c))
        off += c
    mesh = plsc.VectorSubcoreMesh(core_axis_name="c", subcore_axis_name="s")

    @functools.partial(
        pl.kernel,
        mesh=mesh,
        out_type=jax.ShapeDtypeStruct((batch, seq, dim), jnp.float32),
        scratch_types=[
            pltpu.VMEM((2, win_ids), jnp.int32),
            pltpu.VMEM((2, win_ids, dim), jnp.float32),
            pltpu.SemaphoreType.DMA((2,)),
            pltpu.SemaphoreType.DMA((2,)),
            pltpu.SemaphoreType.DMA((2,)),
        ],
        compiler_params=pltpu.CompilerParams(use_tc_tiling_on_sc=False),
    )
    def gather_kernel(idx_hbm, table_hbm, out_hbm, idx_v, rows_v, sem_i,
                      sem_g, sem_o):
        wid = lax.axis_index("s") * _NC + lax.axis_index("c")
        id_base = wid * ids_per_worker
        batch_base = wid * batches_per_worker

        def issue_idx(w, b):
            pltpu.async_copy(
                idx_hbm.at[pl.ds(id_base + w * win_ids, win_ids)],
                idx_v.at[b], sem_i.at[b])

        def wait_idx(b):
            pltpu.make_async_copy(
                idx_hbm.at[pl.ds(0, win_ids)], idx_v.at[b], sem_i.at[b]).wait()

        def issue_gathers(b):
            for (o, c) in chunks:
                pltpu.async_copy(
                    table_hbm.at[idx_v.at[b, pl.ds(o, c)]],
                    rows_v.at[b, pl.ds(o, c)], sem_g.at[b])

        def wait_gathers(b):
            for (o, c) in chunks:
                pltpu.make_async_copy(
                    table_hbm.at[pl.ds(0, c)],
                    rows_v.at[b, pl.ds(o, c)], sem_g.at[b]).wait()

        def issue_out(w, b):
            for j in range(_BPW):
                pltpu.async_copy(
                    rows_v.at[b, pl.ds(j * seq, seq)],
                    out_hbm.at[batch_base + w * _BPW + j], sem_o.at[b])

        def wait_out(b):
            for j in range(_BPW):
                pltpu.make_async_copy(
                    rows_v.at[b, pl.ds(j * seq, seq)], out_hbm.at[0],
                    sem_o.at[b]).wait()

        issue_idx(0, 0)
        issue_idx(1, 1)
        wait_idx(0)
        issue_gathers(0)

        @pl.loop(0, n_win - 2, step=2)
        def _(g):
            wait_idx(1)

            @pl.when(g > 0)
            def _():
                wait_out(1)

            issue_gathers(1)
            wait_gathers(0)
            issue_out(g, 0)
            issue_idx(g + 2, 0)

            wait_idx(0)
            wait_out(0)
            issue_gathers(0)
            wait_gathers(1)
            issue_out(g + 1, 1)
            issue_idx(g + 3, 1)

        wait_idx(1)
        wait_out(1)
        issue_gathers(1)
        wait_gathers(0)
        issue_out(n_win - 2, 0)
        wait_gathers(1)
        issue_out(n_win - 1, 1)
        wait_out(0)
        wait_out(1)

    return gather_kernel(idx_flat, weight)


def kernel(x, weight):
    b, s = x.shape
    idx_flat = x.reshape(b * s).astype(jnp.int32)
    return _sc_gather(idx_flat, weight, b, s)
